# baseline (speedup 1.0000x reference)
import jax
import jax.numpy as jnp
import numpy as np
from jax import lax
from jax.experimental import pallas as pl
from jax.experimental.pallas import tpu as pltpu
from jax.experimental.pallas import tpu_sc as plsc

N = 10000
E = 320000
F = 128

NWORK = 32
CHUNK = 80
KCH = 128
EPAD = NWORK * KCH * CHUNK
TRASH = N
AGG_ROWS = 10240
ZROWS = 160

ROW_B = 1000
DEG_W = F


def _poly_coeffs(d: int = 4) -> np.ndarray:
    import math
    P = np.polynomial.polynomial
    c = np.zeros(d * (d + 1) + 1, dtype=np.float64)
    for i in range(d + 1):
        t = P.polymul(P.polypow(np.array([0.0, 0.5]), i),
                      P.polypow(np.array([1.0, -0.5]), d - i))
        beta = math.gamma(i + 1) * math.gamma(d + 1 - i) / math.gamma(d + 2)
        t = np.asarray(t, dtype=np.float64) / beta
        for k in range(t.shape[0]):
            c[d * i + k] += t[k]
    return c


COEFFS = _poly_coeffs(4)


BCH = 16
NB = KCH // BCH
RING = 32
ZB = 80


def _sc_agg_body(table, gidx, sidx, zeros_hbm, out, gb, db, rows0, rows1,
                 rows2, rows3, agg, gsem0, gsem1, gsem2, gsem3, ssem0, ssem1,
                 ssem2, ssem3, isem):
    cid = lax.axis_index("c")
    sid = lax.axis_index("s")
    wid = sid * 2 + cid
    base = wid * NB
    rowbufs = (rows0, rows1, rows2, rows3)
    gsems = (gsem0, gsem1, gsem2, gsem3)
    ssems = (ssem0, ssem1, ssem2, ssem3)

    def refill(i, half, start):
        g = pltpu.make_async_copy(gidx.at[base + i],
                                  gb.at[pl.ds(half * 16, BCH)], isem)
        d = pltpu.make_async_copy(sidx.at[base + i],
                                  db.at[pl.ds(half * 16, BCH)], isem)
        if start:
            g.start()
            d.start()
        else:
            g.wait()
            d.wait()

    def gather(slot, b, start):
        cp = pltpu.make_async_copy(table.at[gb.at[slot]], rowbufs[b],
                                   gsems[b])
        cp.start() if start else cp.wait()

    def scatter(slot, b, start):
        cp = pltpu.make_async_copy(rowbufs[b], agg.at[db.at[slot]],
                                   ssems[b])
        cp.start(add=True) if start else cp.wait()

    pltpu.sync_copy(zeros_hbm, rows0.at[pl.ds(0, ZB)])
    for i in range(8):
        pltpu.sync_copy(rows0.at[pl.ds(0, ZB)],
                        agg.at[pl.ds(sid * 640 + i * ZB, ZB)])
    plsc.subcore_barrier()

    pltpu.sync_copy(gidx.at[base], gb.at[pl.ds(0, BCH)])
    pltpu.sync_copy(sidx.at[base], db.at[pl.ds(0, BCH)])
    gather(jnp.int32(0), 0, True)

    def batch(i, carry):
        par = lax.rem(i, 2)
        off = par * 16
        oth = (1 - par) * 16
        for jj in range(BCH):
            b = jj % 4
            if jj >= 3:
                scatter(off + jj - 3, (b + 1) % 4, False)
            else:
                @pl.when(i > 0)
                def _():
                    scatter(oth + BCH + jj - 3, (b + 1) % 4, False)
            if jj == 2:
                @pl.when(i + 1 < NB)
                def _():
                    refill(i + 1, 1 - par, True)
            if jj < BCH - 1:
                gather(off + jj + 1, (b + 1) % 4, True)
            else:
                @pl.when(i + 1 < NB)
                def _():
                    refill(i + 1, 1 - par, False)
                    gather(oth, (b + 1) % 4, True)
            gather(off + jj, b, False)
            scatter(off + jj, b, True)
        return carry

    lax.fori_loop(0, NB, batch, 0)
    _lpar = (NB - 1) % 2
    for jj in (BCH - 3, BCH - 2, BCH - 1):
        scatter(jnp.int32(_lpar * 16 + jj), jj % 4, False)
    plsc.subcore_barrier()

    for i in range(8):
        off = sid * 640 + i * ZB
        pltpu.sync_copy(agg.at[pl.ds(off, ZB)], rows0.at[pl.ds(0, ZB)])
        pltpu.sync_copy(rows0.at[pl.ds(0, ZB)], out.at[cid, pl.ds(off, ZB)])


@jax.jit
def _sc_agg(table, gidx, sidx):
    zeros_hbm = jnp.zeros((ZB, F), jnp.float32)
    mesh = plsc.VectorSubcoreMesh(core_axis_name="c", subcore_axis_name="s")
    return pl.kernel(
        _sc_agg_body,
        mesh=mesh,
        out_type=jax.ShapeDtypeStruct((2, AGG_ROWS, F), jnp.float32),
        scratch_types=[
            pltpu.VMEM((RING, CHUNK), jnp.int32),
            pltpu.VMEM((RING, CHUNK), jnp.int32),
            pltpu.VMEM((CHUNK, F), jnp.float32),
            pltpu.VMEM((CHUNK, F), jnp.float32),
            pltpu.VMEM((CHUNK, F), jnp.float32),
            pltpu.VMEM((CHUNK, F), jnp.float32),
            pltpu.VMEM_SHARED((AGG_ROWS, F), jnp.float32),
            pltpu.SemaphoreType.DMA,
            pltpu.SemaphoreType.DMA,
            pltpu.SemaphoreType.DMA,
            pltpu.SemaphoreType.DMA,
            pltpu.SemaphoreType.DMA,
            pltpu.SemaphoreType.DMA,
            pltpu.SemaphoreType.DMA,
            pltpu.SemaphoreType.DMA,
            pltpu.SemaphoreType.DMA,
        ],
    )(table, gidx, sidx, zeros_hbm)


def _mlp_body(x_ref, w1_ref, b1_ref, w2_ref, b2_ref, o_ref):
    dn = (((1,), (1,)), ((), ()))
    h = lax.dot_general(x_ref[...], w1_ref[...], dn,
                        preferred_element_type=jnp.float32) + b1_ref[...]
    h = jnp.maximum(h, 0.0)
    h = lax.dot_general(h, w2_ref[...], dn,
                        preferred_element_type=jnp.float32) + b2_ref[...]
    o_ref[...] = jnp.maximum(h, 0.0)


@jax.jit
def _mlp(x, W1, b1, W2, b2):
    g = N // ROW_B
    return pl.pallas_call(
        _mlp_body,
        grid=(g,),
        in_specs=[
            pl.BlockSpec((ROW_B, F), lambda i: (i, 0)),
            pl.BlockSpec((2 * F, F), lambda i: (0, 0)),
            pl.BlockSpec((1, 2 * F), lambda i: (0, 0)),
            pl.BlockSpec((F, 2 * F), lambda i: (0, 0)),
            pl.BlockSpec((1, F), lambda i: (0, 0)),
        ],
        out_specs=pl.BlockSpec((ROW_B, F), lambda i: (i, 0)),
        out_shape=jax.ShapeDtypeStruct((N, F), jnp.float32),
    )(x, W1, b1.reshape(1, -1), W2, b2.reshape(1, -1))


def _prep_body(h_ref, degp_ref, v0_ref, acc0_ref, dinv2_ref, dsq_ref):
    degp = degp_ref[...]
    deg = degp[0, :, 0:1] + degp[1, :, 0:1]
    degc = jnp.maximum(deg, 1.0)
    dinv2_ref[...] = 1.0 / degc
    dsq_ref[...] = jnp.sqrt(degc)
    v0 = h_ref[...] * lax.rsqrt(degc)
    v0_ref[...] = v0
    acc0_ref[...] = jnp.float32(COEFFS[0]) * v0


@jax.jit
def _prep(h, degp):
    g = N // ROW_B
    return pl.pallas_call(
        _prep_body,
        grid=(g,),
        in_specs=[
            pl.BlockSpec((ROW_B, F), lambda i: (i, 0)),
            pl.BlockSpec((2, ROW_B, DEG_W), lambda i: (0, i, 0)),
        ],
        out_specs=[
            pl.BlockSpec((ROW_B, F), lambda i: (i, 0)),
            pl.BlockSpec((ROW_B, F), lambda i: (i, 0)),
            pl.BlockSpec((ROW_B, 1), lambda i: (i, 0)),
            pl.BlockSpec((ROW_B, 1), lambda i: (i, 0)),
        ],
        out_shape=[
            jax.ShapeDtypeStruct((N, F), jnp.float32),
            jax.ShapeDtypeStruct((N, F), jnp.float32),
            jax.ShapeDtypeStruct((N, 1), jnp.float32),
            jax.ShapeDtypeStruct((N, 1), jnp.float32),
        ],
    )(h, degp)


def _comb_body(c_ref, v_ref, aggp_ref, dinv2_ref, acc_ref, vout_ref,
               accout_ref):
    aggp = aggp_ref[...]
    agg = aggp[0] + aggp[1]
    vnew = v_ref[...] - dinv2_ref[...] * agg
    vout_ref[...] = vnew
    accout_ref[...] = acc_ref[...] + c_ref[0] * vnew


@jax.jit
def _comb(c, v, aggp, dinv2, acc):
    g = N // ROW_B
    return pl.pallas_call(
        _comb_body,
        grid_spec=pltpu.PrefetchScalarGridSpec(
            num_scalar_prefetch=1,
            grid=(g,),
            in_specs=[
                pl.BlockSpec((ROW_B, F), lambda i, c: (i, 0)),
                pl.BlockSpec((2, ROW_B, F), lambda i, c: (0, i, 0)),
                pl.BlockSpec((ROW_B, 1), lambda i, c: (i, 0)),
                pl.BlockSpec((ROW_B, F), lambda i, c: (i, 0)),
            ],
            out_specs=[
                pl.BlockSpec((ROW_B, F), lambda i, c: (i, 0)),
                pl.BlockSpec((ROW_B, F), lambda i, c: (i, 0)),
            ],
        ),
        out_shape=[
            jax.ShapeDtypeStruct((N, F), jnp.float32),
            jax.ShapeDtypeStruct((N, F), jnp.float32),
        ],
    )(c, v, aggp, dinv2, acc)


def _scale_body(acc_ref, dsq_ref, o_ref):
    o_ref[...] = acc_ref[...] * dsq_ref[...]


@jax.jit
def _scale(acc, dsq):
    g = N // ROW_B
    return pl.pallas_call(
        _scale_body,
        grid=(g,),
        in_specs=[
            pl.BlockSpec((ROW_B, F), lambda i: (i, 0)),
            pl.BlockSpec((ROW_B, 1), lambda i: (i, 0)),
        ],
        out_specs=pl.BlockSpec((ROW_B, F), lambda i: (i, 0)),
        out_shape=jax.ShapeDtypeStruct((N, F), jnp.float32),
    )(acc, dsq)


def _gram_body(a_ref, b_ref, o_ref):
    o_ref[...] = lax.dot_general(a_ref[...], b_ref[...],
                                 (((1,), (1,)), ((), ())),
                                 preferred_element_type=jnp.float32)


GRAM_B = 400


@jax.jit
def _gram(a):
    g = N // GRAM_B
    return pl.pallas_call(
        _gram_body,
        grid=(g,),
        in_specs=[
            pl.BlockSpec((GRAM_B, F), lambda i: (i, 0)),
            pl.BlockSpec((N, F), lambda i: (0, 0)),
        ],
        out_specs=pl.BlockSpec((GRAM_B, N), lambda i: (i, 0)),
        out_shape=jax.ShapeDtypeStruct((N, N), jnp.float32),
        compiler_params=pltpu.CompilerParams(
            dimension_semantics=("arbitrary",)),
    )(a, a)


def kernel(in_feat, edge_index, dst_nodes, W1, b1, W2, b2):
    src = edge_index[0]
    dst = edge_index[1]
    npad = EPAD - E
    pad0 = jnp.zeros((npad,), jnp.int32)
    padT = jnp.full((npad,), TRASH, jnp.int32)
    src_g = jnp.concatenate([src, pad0]).reshape(NWORK * NB, BCH, CHUNK)
    src_s = jnp.concatenate([src, padT]).reshape(NWORK * NB, BCH, CHUNK)
    dst_s = jnp.concatenate([dst, padT]).reshape(NWORK * NB, BCH, CHUNK)

    h = _mlp(in_feat, W1, b1, W2, b2)

    ones_table = jnp.ones((N, F), jnp.float32)
    degp = _sc_agg(ones_table, src_g, src_s)

    v, acc, dinv2, dsq = _prep(h, degp)

    for j in range(1, len(COEFFS)):
        aggp = _sc_agg(v, src_g, dst_s)
        cj = jnp.full((1,), COEFFS[j], jnp.float32)
        v, acc = _comb(cj, v, aggp, dinv2, acc)

    all_h = _scale(acc, dsq)
    all_h = jnp.take(all_h, dst_nodes, axis=0)
    recons = _gram(all_h)
    return recons, all_h


# speedup vs baseline: 2.1243x; 2.1243x over previous
import jax
import jax.numpy as jnp
import numpy as np
from jax import lax
from jax.experimental import pallas as pl
from jax.experimental.pallas import tpu as pltpu
from jax.experimental.pallas import tpu_sc as plsc

N = 10000
E = 320000
F = 128

NWORK = 32
CHUNK = 96
KCH = 105
EPAD = NWORK * KCH * CHUNK
TRASH = N
AGG_ROWS = 10240
ZROWS = 160

ROW_B = 1000
DEG_W = F


def _poly_coeffs(d: int = 4) -> np.ndarray:
    import math
    P = np.polynomial.polynomial
    c = np.zeros(d * (d + 1) + 1, dtype=np.float64)
    for i in range(d + 1):
        t = P.polymul(P.polypow(np.array([0.0, 0.5]), i),
                      P.polypow(np.array([1.0, -0.5]), d - i))
        beta = math.gamma(i + 1) * math.gamma(d + 1 - i) / math.gamma(d + 2)
        t = np.asarray(t, dtype=np.float64) / beta
        for k in range(t.shape[0]):
            c[d * i + k] += t[k]
    return c


COEFFS = _poly_coeffs(4)


BCH = 15
NB = KCH // BCH
RING = 32
ZB = 80


def _sc_agg_body(table, gidx, sidx, zeros_hbm, out, gb, db, rows0, rows1,
                 rows2, agg, gsem0, gsem1, gsem2, ssem0, ssem1, ssem2, isem):
    cid = lax.axis_index("c")
    sid = lax.axis_index("s")
    wid = sid * 2 + cid
    base = wid * NB
    rowbufs = (rows0, rows1, rows2)
    gsems = (gsem0, gsem1, gsem2)
    ssems = (ssem0, ssem1, ssem2)

    def refill(i, half, start):
        g = pltpu.make_async_copy(gidx.at[base + i],
                                  gb.at[pl.ds(half * 16, BCH)], isem)
        d = pltpu.make_async_copy(sidx.at[base + i],
                                  db.at[pl.ds(half * 16, BCH)], isem)
        if start:
            g.start()
            d.start()
        else:
            g.wait()
            d.wait()

    def gather(slot, b, start):
        cp = pltpu.make_async_copy(table.at[gb.at[slot]], rowbufs[b],
                                   gsems[b])
        cp.start() if start else cp.wait()

    def scatter(slot, b, start):
        cp = pltpu.make_async_copy(rowbufs[b], agg.at[db.at[slot]],
                                   ssems[b])
        cp.start(add=True) if start else cp.wait()

    pltpu.sync_copy(zeros_hbm, rows0.at[pl.ds(0, ZB)])
    for i in range(8):
        pltpu.sync_copy(rows0.at[pl.ds(0, ZB)],
                        agg.at[pl.ds(sid * 640 + i * ZB, ZB)])
    plsc.subcore_barrier()

    pltpu.sync_copy(gidx.at[base], gb.at[pl.ds(0, BCH)])
    pltpu.sync_copy(sidx.at[base], db.at[pl.ds(0, BCH)])
    gather(jnp.int32(0), 0, True)

    def batch(i, carry):
        par = lax.rem(i, 2)
        off = par * 16
        oth = (1 - par) * 16
        for jj in range(BCH):
            b = jj % 3
            if jj >= 2:
                scatter(off + jj - 2, (b + 1) % 3, False)
            else:
                @pl.when(i > 0)
                def _():
                    scatter(oth + BCH + jj - 2, (b + 1) % 3, False)
            if jj == 1:
                @pl.when(i + 1 < NB)
                def _():
                    refill(i + 1, 1 - par, True)
            if jj < BCH - 1:
                gather(off + jj + 1, (b + 1) % 3, True)
            else:
                @pl.when(i + 1 < NB)
                def _():
                    refill(i + 1, 1 - par, False)
                    gather(oth, (b + 1) % 3, True)
            gather(off + jj, b, False)
            scatter(off + jj, b, True)
        return carry

    lax.fori_loop(0, NB, batch, 0)
    _lpar = (NB - 1) % 2
    for jj in (BCH - 2, BCH - 1):
        scatter(jnp.int32(_lpar * 16 + jj), jj % 3, False)
    plsc.subcore_barrier()

    for i in range(8):
        off = sid * 640 + i * ZB
        pltpu.sync_copy(agg.at[pl.ds(off, ZB)], rows0.at[pl.ds(0, ZB)])
        pltpu.sync_copy(rows0.at[pl.ds(0, ZB)], out.at[cid, pl.ds(off, ZB)])


@jax.jit
def _sc_agg(table, gidx, sidx):
    zeros_hbm = jnp.zeros((ZB, F), jnp.float32)
    mesh = plsc.VectorSubcoreMesh(core_axis_name="c", subcore_axis_name="s")
    return pl.kernel(
        _sc_agg_body,
        mesh=mesh,
        out_type=jax.ShapeDtypeStruct((2, AGG_ROWS, F), jnp.float32),
        scratch_types=[
            pltpu.VMEM((RING, CHUNK), jnp.int32),
            pltpu.VMEM((RING, CHUNK), jnp.int32),
            pltpu.VMEM((CHUNK, F), jnp.float32),
            pltpu.VMEM((CHUNK, F), jnp.float32),
            pltpu.VMEM((CHUNK, F), jnp.float32),
            pltpu.VMEM_SHARED((AGG_ROWS, F), jnp.float32),
            pltpu.SemaphoreType.DMA,
            pltpu.SemaphoreType.DMA,
            pltpu.SemaphoreType.DMA,
            pltpu.SemaphoreType.DMA,
            pltpu.SemaphoreType.DMA,
            pltpu.SemaphoreType.DMA,
            pltpu.SemaphoreType.DMA,
        ],
    )(table, gidx, sidx, zeros_hbm)


def _mlp_body(x_ref, w1_ref, b1_ref, w2_ref, b2_ref, o_ref):
    dn = (((1,), (1,)), ((), ()))
    h = lax.dot_general(x_ref[...], w1_ref[...], dn,
                        preferred_element_type=jnp.float32) + b1_ref[...]
    h = jnp.maximum(h, 0.0)
    h = lax.dot_general(h, w2_ref[...], dn,
                        preferred_element_type=jnp.float32) + b2_ref[...]
    o_ref[...] = jnp.maximum(h, 0.0)


@jax.jit
def _mlp(x, W1, b1, W2, b2):
    g = N // ROW_B
    return pl.pallas_call(
        _mlp_body,
        grid=(g,),
        in_specs=[
            pl.BlockSpec((ROW_B, F), lambda i: (i, 0)),
            pl.BlockSpec((2 * F, F), lambda i: (0, 0)),
            pl.BlockSpec((1, 2 * F), lambda i: (0, 0)),
            pl.BlockSpec((F, 2 * F), lambda i: (0, 0)),
            pl.BlockSpec((1, F), lambda i: (0, 0)),
        ],
        out_specs=pl.BlockSpec((ROW_B, F), lambda i: (i, 0)),
        out_shape=jax.ShapeDtypeStruct((N, F), jnp.float32),
    )(x, W1, b1.reshape(1, -1), W2, b2.reshape(1, -1))


def _prep_body(h_ref, degp_ref, v0_ref, acc0_ref, dinv2_ref, dsq_ref):
    degp = degp_ref[...]
    deg = degp[0, :, 0:1] + degp[1, :, 0:1]
    degc = jnp.maximum(deg, 1.0)
    dinv2_ref[...] = 1.0 / degc
    dsq_ref[...] = jnp.sqrt(degc)
    v0 = h_ref[...] * lax.rsqrt(degc)
    v0_ref[...] = v0
    acc0_ref[...] = jnp.float32(COEFFS[0]) * v0


@jax.jit
def _prep(h, degp):
    g = N // ROW_B
    return pl.pallas_call(
        _prep_body,
        grid=(g,),
        in_specs=[
            pl.BlockSpec((ROW_B, F), lambda i: (i, 0)),
            pl.BlockSpec((2, ROW_B, DEG_W), lambda i: (0, i, 0)),
        ],
        out_specs=[
            pl.BlockSpec((ROW_B, F), lambda i: (i, 0)),
            pl.BlockSpec((ROW_B, F), lambda i: (i, 0)),
            pl.BlockSpec((ROW_B, 1), lambda i: (i, 0)),
            pl.BlockSpec((ROW_B, 1), lambda i: (i, 0)),
        ],
        out_shape=[
            jax.ShapeDtypeStruct((N, F), jnp.float32),
            jax.ShapeDtypeStruct((N, F), jnp.float32),
            jax.ShapeDtypeStruct((N, 1), jnp.float32),
            jax.ShapeDtypeStruct((N, 1), jnp.float32),
        ],
    )(h, degp)


def _comb_body(c_ref, v_ref, aggp_ref, dinv2_ref, acc_ref, vout_ref,
               accout_ref):
    aggp = aggp_ref[...]
    agg = aggp[0] + aggp[1]
    vnew = v_ref[...] - dinv2_ref[...] * agg
    vout_ref[...] = vnew
    accout_ref[...] = acc_ref[...] + c_ref[0] * vnew


@jax.jit
def _comb(c, v, aggp, dinv2, acc):
    g = N // ROW_B
    return pl.pallas_call(
        _comb_body,
        grid_spec=pltpu.PrefetchScalarGridSpec(
            num_scalar_prefetch=1,
            grid=(g,),
            in_specs=[
                pl.BlockSpec((ROW_B, F), lambda i, c: (i, 0)),
                pl.BlockSpec((2, ROW_B, F), lambda i, c: (0, i, 0)),
                pl.BlockSpec((ROW_B, 1), lambda i, c: (i, 0)),
                pl.BlockSpec((ROW_B, F), lambda i, c: (i, 0)),
            ],
            out_specs=[
                pl.BlockSpec((ROW_B, F), lambda i, c: (i, 0)),
                pl.BlockSpec((ROW_B, F), lambda i, c: (i, 0)),
            ],
        ),
        out_shape=[
            jax.ShapeDtypeStruct((N, F), jnp.float32),
            jax.ShapeDtypeStruct((N, F), jnp.float32),
        ],
    )(c, v, aggp, dinv2, acc)


def _scale_body(acc_ref, dsq_ref, o_ref):
    o_ref[...] = acc_ref[...] * dsq_ref[...]


@jax.jit
def _scale(acc, dsq):
    g = N // ROW_B
    return pl.pallas_call(
        _scale_body,
        grid=(g,),
        in_specs=[
            pl.BlockSpec((ROW_B, F), lambda i: (i, 0)),
            pl.BlockSpec((ROW_B, 1), lambda i: (i, 0)),
        ],
        out_specs=pl.BlockSpec((ROW_B, F), lambda i: (i, 0)),
        out_shape=jax.ShapeDtypeStruct((N, F), jnp.float32),
    )(acc, dsq)


def _gram_body(a_ref, b_ref, o_ref):
    o_ref[...] = lax.dot_general(a_ref[...], b_ref[...],
                                 (((1,), (1,)), ((), ())),
                                 preferred_element_type=jnp.float32)


GRAM_B = 400


@jax.jit
def _gram(a):
    g = N // GRAM_B
    return pl.pallas_call(
        _gram_body,
        grid=(g,),
        in_specs=[
            pl.BlockSpec((GRAM_B, F), lambda i: (i, 0)),
            pl.BlockSpec((N, F), lambda i: (0, 0)),
        ],
        out_specs=pl.BlockSpec((GRAM_B, N), lambda i: (i, 0)),
        out_shape=jax.ShapeDtypeStruct((N, N), jnp.float32),
        compiler_params=pltpu.CompilerParams(
            dimension_semantics=("arbitrary",)),
    )(a, a)


def kernel(in_feat, edge_index, dst_nodes, W1, b1, W2, b2):
    src = edge_index[0]
    dst = edge_index[1]
    npad = EPAD - E
    pad0 = jnp.zeros((npad,), jnp.int32)
    padT = jnp.full((npad,), TRASH, jnp.int32)
    src_g = jnp.concatenate([src, pad0]).reshape(NWORK * NB, BCH, CHUNK)
    src_s = jnp.concatenate([src, padT]).reshape(NWORK * NB, BCH, CHUNK)
    dst_s = jnp.concatenate([dst, padT]).reshape(NWORK * NB, BCH, CHUNK)

    h = _mlp(in_feat, W1, b1, W2, b2)

    ones_table = jnp.ones((N, F), jnp.float32)
    degp = _sc_agg(ones_table, src_g, src_s)

    v, acc, dinv2, dsq = _prep(h, degp)

    for j in range(1, len(COEFFS)):
        aggp = _sc_agg(v, src_g, dst_s)
        cj = jnp.full((1,), COEFFS[j], jnp.float32)
        v, acc = _comb(cj, v, aggp, dinv2, acc)

    all_h = _scale(acc, dsq)
    all_h = jnp.take(all_h, dst_nodes, axis=0)
    recons = _gram(all_h)
    return recons, all_h
